# baseline (speedup 1.0000x reference)
import dataclasses
import functools

import jax
import jax.numpy as jnp
from jax import lax
from jax.experimental import pallas as pl
from jax.experimental.pallas import tpu as pltpu
from jax.experimental.pallas import tpu_sc as plsc

GAMMA = 12.0
NC = 2
NS = 16
NW = NC * NS
LANES = 16
IDX_CHUNK = 128


@functools.lru_cache(maxsize=None)
def _build(B, D):
    assert B % (NW * LANES) == 0 and D % LANES == 0
    bpw = B // NW
    nchunk = bpw // IDX_CHUNK
    assert nchunk == 4

    mesh = plsc.VectorSubcoreMesh(core_axis_name="c", subcore_axis_name="s")

    cp = pltpu.CompilerParams()
    if "needs_layout_passes" in pltpu.CompilerParams.__dataclass_fields__:
        cp = dataclasses.replace(cp, needs_layout_passes=False)
    if "use_tc_tiling_on_sc" in pltpu.CompilerParams.__dataclass_fields__:
        cp = dataclasses.replace(cp, use_tc_tiling_on_sc=False)
    if "skip_device_barrier" in pltpu.CompilerParams.__dataclass_fields__:
        cp = dataclasses.replace(cp, skip_device_barrier=True)

    @functools.partial(
        pl.kernel,
        mesh=mesh,
        compiler_params=cp,
        out_type=jax.ShapeDtypeStruct((B,), jnp.float32),
        scratch_types=[
            pltpu.VMEM((16, IDX_CHUNK), jnp.int32),
            pltpu.VMEM((bpw, D), jnp.float32),
            pltpu.VMEM((bpw, D), jnp.float32),
            pltpu.VMEM((bpw, D), jnp.float32),
            pltpu.VMEM((LANES, LANES), jnp.float32),
            pltpu.VMEM((bpw,), jnp.float32),
            pltpu.SemaphoreType.DMA,
        ],
    )
    def kge(ent_hbm, rel_hbm, ids_hbm, out_hbm,
            ix_v, h_v, r_v, t_v, p_v, s_v, sem):
        wid = lax.axis_index("s") * NC + lax.axis_index("c")
        pltpu.sync_copy(ids_hbm.at[wid], ix_v)

        iota16 = lax.iota(jnp.int32, 16)

        for k in range(nchunk):
            dst = pl.ds(k * IDX_CHUNK, IDX_CHUNK)
            pltpu.async_copy(ent_hbm.at[ix_v.at[k]], h_v.at[dst], sem)
            pltpu.async_copy(rel_hbm.at[ix_v.at[nchunk + k]], r_v.at[dst], sem)
            pltpu.async_copy(ent_hbm.at[ix_v.at[2 * nchunk + k]], t_v.at[dst], sem)

        pltpu.make_async_copy(ent_hbm.at[pl.ds(0, bpw)], h_v, sem).wait()
        pltpu.make_async_copy(rel_hbm.at[pl.ds(0, bpw)], r_v, sem).wait()
        pltpu.make_async_copy(ent_hbm.at[pl.ds(0, bpw)], t_v, sem).wait()

        @pl.loop(0, bpw, step=LANES)
        def _group(g):
            @pl.loop(0, LANES)
            def _row(i):
                b = g + i
                acc = jnp.abs(h_v[b, pl.ds(0, LANES)]
                              + r_v[b, pl.ds(0, LANES)]
                              - t_v[b, pl.ds(0, LANES)])
                for c in range(1, D // LANES):
                    sl = pl.ds(c * LANES, LANES)
                    acc = acc + jnp.abs(h_v[b, sl] + r_v[b, sl] - t_v[b, sl])
                p_v[i, :] = acc

            tot = plsc.load_gather(p_v, [iota16, jnp.full((16,), 0, jnp.int32)])
            for j in range(1, LANES):
                tot = tot + plsc.load_gather(p_v, [iota16, jnp.full((16,), j, jnp.int32)])
            s_v[pl.ds(g, LANES)] = GAMMA - tot

        pltpu.sync_copy(s_v, out_hbm.at[pl.ds(wid * bpw, bpw)])

    return kge


def kernel(sample, entity_embedding, relation_embedding):
    B = sample.shape[0]
    D = entity_embedding.shape[1]
    ids = sample.astype(jnp.int32)
    bpw = B // NW
    nchunk = bpw // IDX_CHUNK
    blk = jnp.concatenate(
        [ids[:, 0].reshape(NW, nchunk, IDX_CHUNK),
         ids[:, 1].reshape(NW, nchunk, IDX_CHUNK),
         ids[:, 2].reshape(NW, nchunk, IDX_CHUNK),
         jnp.zeros((NW, 16 - 3 * nchunk, IDX_CHUNK), jnp.int32)], axis=1)
    score = _build(B, D)(entity_embedding, relation_embedding, blk)
    return score.reshape(B, 1)


# speedup vs baseline: 1.0010x; 1.0010x over previous
import dataclasses
import functools

import jax
import jax.numpy as jnp
from jax import lax
from jax.experimental import pallas as pl
from jax.experimental.pallas import tpu as pltpu
from jax.experimental.pallas import tpu_sc as plsc

GAMMA = 12.0
NC = 2
NS = 16
NW = NC * NS
LANES = 16
IDX_CHUNK = 128


def _params():
    cp = pltpu.CompilerParams()
    if "needs_layout_passes" in pltpu.CompilerParams.__dataclass_fields__:
        cp = dataclasses.replace(cp, needs_layout_passes=False)
    if "use_tc_tiling_on_sc" in pltpu.CompilerParams.__dataclass_fields__:
        cp = dataclasses.replace(cp, use_tc_tiling_on_sc=False)
    return cp


@functools.lru_cache(maxsize=None)
def _build_ht(B, D):
    bpw = B // NW
    nchunk = bpw // IDX_CHUNK
    mesh = plsc.VectorSubcoreMesh(core_axis_name="c", subcore_axis_name="s")

    @functools.partial(
        pl.kernel,
        mesh=mesh,
        compiler_params=_params(),
        out_type=jax.ShapeDtypeStruct((B, D), jnp.float32),
        scratch_types=[
            pltpu.VMEM((8, IDX_CHUNK), jnp.int32),
            pltpu.VMEM((bpw, D), jnp.float32),
            pltpu.VMEM((bpw, D), jnp.float32),
            pltpu.SemaphoreType.DMA,
        ],
    )
    def ht(ent_hbm, ids_hbm, dt_hbm, ix_v, h_v, t_v, sem):
        wid = lax.axis_index("s") * NC + lax.axis_index("c")
        pltpu.sync_copy(ids_hbm.at[wid], ix_v)

        for k in range(nchunk):
            dst = pl.ds(k * IDX_CHUNK, IDX_CHUNK)
            pltpu.async_copy(ent_hbm.at[ix_v.at[k]], h_v.at[dst], sem)
            pltpu.async_copy(ent_hbm.at[ix_v.at[nchunk + k]], t_v.at[dst], sem)
        pltpu.make_async_copy(ent_hbm.at[pl.ds(0, bpw)], h_v, sem).wait()
        pltpu.make_async_copy(ent_hbm.at[pl.ds(0, bpw)], t_v, sem).wait()

        @pl.loop(0, bpw)
        def _rows(b):
            for c in range(D // LANES):
                sl = pl.ds(c * LANES, LANES)
                h_v[b, sl] = h_v[b, sl] - t_v[b, sl]

        pltpu.sync_copy(h_v, dt_hbm.at[pl.ds(wid * bpw, bpw)])

    return ht


@functools.lru_cache(maxsize=None)
def _build_score(B, D):
    bpw = B // NW
    nchunk = bpw // IDX_CHUNK
    mesh = plsc.VectorSubcoreMesh(core_axis_name="c", subcore_axis_name="s")

    @functools.partial(
        pl.kernel,
        mesh=mesh,
        compiler_params=_params(),
        out_type=jax.ShapeDtypeStruct((B,), jnp.float32),
        scratch_types=[
            pltpu.VMEM((4, IDX_CHUNK), jnp.int32),
            pltpu.VMEM((bpw, D), jnp.float32),
            pltpu.VMEM((bpw, D), jnp.float32),
            pltpu.VMEM((LANES, LANES), jnp.float32),
            pltpu.VMEM((bpw,), jnp.float32),
            pltpu.SemaphoreType.DMA,
        ],
    )
    def score(rel_hbm, dt_hbm, ids_hbm, out_hbm,
              ix_v, r_v, d_v, p_v, s_v, sem):
        wid = lax.axis_index("s") * NC + lax.axis_index("c")
        pltpu.sync_copy(ids_hbm.at[wid], ix_v)

        pltpu.async_copy(dt_hbm.at[pl.ds(wid * bpw, bpw)], d_v, sem)
        for k in range(nchunk):
            dst = pl.ds(k * IDX_CHUNK, IDX_CHUNK)
            pltpu.async_copy(rel_hbm.at[ix_v.at[k]], r_v.at[dst], sem)
        pltpu.make_async_copy(rel_hbm.at[pl.ds(0, bpw)], r_v, sem).wait()
        pltpu.make_async_copy(rel_hbm.at[pl.ds(0, bpw)], d_v, sem).wait()

        iota16 = lax.iota(jnp.int32, 16)

        @pl.loop(0, bpw, step=LANES)
        def _group(g):
            @pl.loop(0, LANES)
            def _row(i):
                b = g + i
                acc = jnp.abs(d_v[b, pl.ds(0, LANES)] + r_v[b, pl.ds(0, LANES)])
                for c in range(1, D // LANES):
                    sl = pl.ds(c * LANES, LANES)
                    acc = acc + jnp.abs(d_v[b, sl] + r_v[b, sl])
                p_v[i, :] = acc

            tot = plsc.load_gather(p_v, [iota16, jnp.full((16,), 0, jnp.int32)])
            for j in range(1, LANES):
                tot = tot + plsc.load_gather(p_v, [iota16, jnp.full((16,), j, jnp.int32)])
            s_v[pl.ds(g, LANES)] = GAMMA - tot

        pltpu.sync_copy(s_v, out_hbm.at[pl.ds(wid * bpw, bpw)])

    return score


def kernel(sample, entity_embedding, relation_embedding):
    B = sample.shape[0]
    D = entity_embedding.shape[1]
    ids = sample.astype(jnp.int32)
    bpw = B // NW
    nchunk = bpw // IDX_CHUNK
    ids_ht = jnp.concatenate(
        [ids[:, 0].reshape(NW, nchunk, IDX_CHUNK),
         ids[:, 2].reshape(NW, nchunk, IDX_CHUNK)], axis=1)
    ids_r = ids[:, 1].reshape(NW, nchunk, IDX_CHUNK)
    dt = _build_ht(B, D)(entity_embedding, ids_ht)
    score = _build_score(B, D)(relation_embedding, dt, ids_r)
    return score.reshape(B, 1)
